# baseline (speedup 1.0000x reference)
import jax
import jax.numpy as jnp
from jax.experimental import pallas as pl


def _mask_cast_kernel(m_ref, o_ref):
    o_ref[...] = m_ref[...] != 0.0


def kernel(x, v, mask):
    B, one, P = mask.shape
    m2 = mask.reshape(B, P)
    out = pl.pallas_call(
        _mask_cast_kernel,
        out_shape=jax.ShapeDtypeStruct((B, P), jnp.bool_),
    )(m2)
    return (x, v, out.reshape(B, one, P))
